# Initial kernel scaffold; baseline (speedup 1.0000x reference)
import jax
import jax.numpy as jnp
from jax.experimental import pallas as pl


def kernel(frames_inputs, frames_lengths, uttr_input, dialog_lengths, W1, b1, Wc, bc, Wo, bo, Wco, bco, Wres, bres, W2, b2, Wout, bout):
    raise NotImplementedError("write your pallas kernel here")


# speedup vs baseline: 1.0825x; 1.0825x over previous
import jax
import jax.numpy as jnp
from jax.experimental import pallas as pl
from jax.experimental.pallas import tpu as pltpu

_B, _T, _D, _H, _C = 64, 512, 256, 128, 7
_F = 4


def _rep_kernel(len_ref, frames_ref, W1_ref, b1_ref, out_ref):
    b = pl.program_id(0)
    L = len_ref[b]
    x = frames_ref[0]
    h = jnp.maximum(
        jnp.dot(x, W1_ref[...], preferred_element_type=jnp.float32) + b1_ref[...],
        0.0,
    )
    t = jax.lax.broadcasted_iota(jnp.int32, (1, _T), 1)
    w = jnp.zeros((1, _T), jnp.float32)
    for k in range(_F + 1):
        tk = t + k
        w = w + jnp.where(
            tk < L, 1.0 / jnp.minimum(tk + 1, _F + 1).astype(jnp.float32), 0.0
        )
    w = w / L.astype(jnp.float32)
    out_ref[0] = jnp.dot(w, h, preferred_element_type=jnp.float32)


def _head_kernel(rep_ref, uttr_ref, dl_ref,
                 Wc_ref, bc_ref, Wo_ref, bo_ref, Wco_ref, bco_ref,
                 Wres_ref, bres_ref, W2_ref, b2_ref, Wout_ref, bout_ref,
                 x_ref, xo_ref, xc_ref, xco_ref):
    rep = rep_ref[...]
    f32 = jnp.float32
    xc_ref[...] = jnp.dot(rep, Wc_ref[...], preferred_element_type=f32) + bc_ref[...]
    xo_ref[...] = jnp.dot(rep, Wo_ref[...], preferred_element_type=f32) + bo_ref[...]
    xco_ref[...] = jnp.dot(rep, Wco_ref[...], preferred_element_type=f32) + bco_ref[...]
    res = jnp.maximum(
        jnp.dot(uttr_ref[...], Wres_ref[...], preferred_element_type=f32)
        + bres_ref[...],
        0.0,
    )
    rep2 = rep + res
    h2 = jnp.maximum(
        jnp.dot(rep2, W2_ref[...], preferred_element_type=f32) + b2_ref[...], 0.0
    )
    dl = dl_ref[...]
    node2 = (h2 * dl) / dl
    x_ref[...] = jnp.dot(node2, Wout_ref[...], preferred_element_type=f32) + bout_ref[...]


def kernel(frames_inputs, frames_lengths, uttr_input, dialog_lengths,
           W1, b1, Wc, bc, Wo, bo, Wco, bco, Wres, bres, W2, b2, Wout, bout):
    lengths = frames_lengths.astype(jnp.int32)
    rep = pl.pallas_call(
        _rep_kernel,
        grid_spec=pltpu.PrefetchScalarGridSpec(
            num_scalar_prefetch=1,
            grid=(_B,),
            in_specs=[
                pl.BlockSpec((1, _T, _D), lambda b, L: (b, 0, 0)),
                pl.BlockSpec((_D, _H), lambda b, L: (0, 0)),
                pl.BlockSpec((1, _H), lambda b, L: (0, 0)),
            ],
            out_specs=pl.BlockSpec((1, 1, _H), lambda b, L: (b, 0, 0)),
        ),
        out_shape=jax.ShapeDtypeStruct((_B, 1, _H), jnp.float32),
    )(lengths, frames_inputs, W1, b1.reshape(1, _H))
    rep = rep.reshape(_B, _H)

    x, xo, xc, xco = pl.pallas_call(
        _head_kernel,
        out_shape=[jax.ShapeDtypeStruct((_B, _C), jnp.float32)] * 4,
    )(rep, uttr_input, dialog_lengths.astype(jnp.float32).reshape(_B, 1),
      Wc, bc.reshape(1, _C), Wo, bo.reshape(1, _C), Wco, bco.reshape(1, _C),
      Wres, bres.reshape(1, _H), W2, b2.reshape(1, _H), Wout, bout.reshape(1, _C))
    return (x, xo, xc, xco)


# speedup vs baseline: 2.3543x; 2.1748x over previous
import jax
import jax.numpy as jnp
from jax.experimental import pallas as pl
from jax.experimental.pallas import tpu as pltpu

_B, _T, _D, _H, _C = 64, 512, 256, 128, 7
_F = 4
_BB = 8


def _rep_kernel(len_ref, frames_ref, W1_ref, b1_ref, out_ref):
    i = pl.program_id(0)
    x = frames_ref[...].reshape(_BB * _T, _D)
    h = jnp.maximum(
        jnp.dot(x, W1_ref[...], preferred_element_type=jnp.float32) + b1_ref[...],
        0.0,
    )
    L = jnp.stack([len_ref[i * _BB + r] for r in range(_BB)]).reshape(_BB, 1)
    col = jax.lax.broadcasted_iota(jnp.int32, (_BB, _BB * _T), 1)
    row = jax.lax.broadcasted_iota(jnp.int32, (_BB, _BB * _T), 0)
    t = col & (_T - 1)
    w = jnp.zeros((_BB, _BB * _T), jnp.float32)
    for k in range(_F + 1):
        tk = t + k
        w = w + jnp.where(
            tk < L, 1.0 / jnp.minimum(tk + 1, _F + 1).astype(jnp.float32), 0.0
        )
    w = jnp.where((col >> 9) == row, w / L.astype(jnp.float32), 0.0)
    out_ref[...] = jnp.dot(w, h, preferred_element_type=jnp.float32)[:, None, :]


def _head_kernel(rep_ref, uttr_ref, dl_ref,
                 Wc_ref, bc_ref, Wo_ref, bo_ref, Wco_ref, bco_ref,
                 Wres_ref, bres_ref, W2_ref, b2_ref, Wout_ref, bout_ref,
                 x_ref, xo_ref, xc_ref, xco_ref):
    rep = rep_ref[...]
    f32 = jnp.float32
    xc_ref[...] = jnp.dot(rep, Wc_ref[...], preferred_element_type=f32) + bc_ref[...]
    xo_ref[...] = jnp.dot(rep, Wo_ref[...], preferred_element_type=f32) + bo_ref[...]
    xco_ref[...] = jnp.dot(rep, Wco_ref[...], preferred_element_type=f32) + bco_ref[...]
    res = jnp.maximum(
        jnp.dot(uttr_ref[...], Wres_ref[...], preferred_element_type=f32)
        + bres_ref[...],
        0.0,
    )
    rep2 = rep + res
    h2 = jnp.maximum(
        jnp.dot(rep2, W2_ref[...], preferred_element_type=f32) + b2_ref[...], 0.0
    )
    dl = dl_ref[...]
    node2 = (h2 * dl) / dl
    x_ref[...] = jnp.dot(node2, Wout_ref[...], preferred_element_type=f32) + bout_ref[...]


def kernel(frames_inputs, frames_lengths, uttr_input, dialog_lengths,
           W1, b1, Wc, bc, Wo, bo, Wco, bco, Wres, bres, W2, b2, Wout, bout):
    lengths = frames_lengths.astype(jnp.int32)
    rep = pl.pallas_call(
        _rep_kernel,
        grid_spec=pltpu.PrefetchScalarGridSpec(
            num_scalar_prefetch=1,
            grid=(_B // _BB,),
            in_specs=[
                pl.BlockSpec((_BB, _T, _D), lambda b, L: (b, 0, 0)),
                pl.BlockSpec((_D, _H), lambda b, L: (0, 0)),
                pl.BlockSpec((1, _H), lambda b, L: (0, 0)),
            ],
            out_specs=pl.BlockSpec((_BB, 1, _H), lambda b, L: (b, 0, 0)),
        ),
        out_shape=jax.ShapeDtypeStruct((_B, 1, _H), jnp.float32),
    )(lengths, frames_inputs, W1, b1.reshape(1, _H))
    rep = rep.reshape(_B, _H)

    x, xo, xc, xco = pl.pallas_call(
        _head_kernel,
        out_shape=[jax.ShapeDtypeStruct((_B, _C), jnp.float32)] * 4,
    )(rep, uttr_input, dialog_lengths.astype(jnp.float32).reshape(_B, 1),
      Wc, bc.reshape(1, _C), Wo, bo.reshape(1, _C), Wco, bco.reshape(1, _C),
      Wres, bres.reshape(1, _H), W2, b2.reshape(1, _H), Wout, bout.reshape(1, _C))
    return (x, xo, xc, xco)
